# Initial kernel scaffold; baseline (speedup 1.0000x reference)
import jax
import jax.numpy as jnp
from jax.experimental import pallas as pl


def kernel(x, edge_index, W1, b1, W2, b2):
    raise NotImplementedError("write your pallas kernel here")


# speedup vs baseline: 59.6691x; 59.6691x over previous
import functools

import jax
import jax.numpy as jnp
from jax import lax
from jax.experimental import pallas as pl
from jax.experimental.pallas import tpu as pltpu
from jax.experimental.pallas import tpu_sc as plsc

_N = 10000
_E = 320000
_D = 128
_H = 16

_NC = 2
_NS = 16
_NW = _NC * _NS

_EPC = _E // _NC
_EPT = _E // _NW
_CHUNK = 2000
_NCHUNK = _EPT // _CHUNK

_NP = 10240
_RPT = _NP // _NS
_DPT = _NP // _NS


def _fill_zero_rows(ref, nrows):
    zero = jnp.zeros((_H,), jnp.float32)

    def body(i, _):
        ref[i, :] = zero
        return 0

    lax.fori_loop(0, nrows, body, 0)


_deg_mesh = plsc.VectorSubcoreMesh(core_axis_name="c", subcore_axis_name="s")


@functools.partial(
    pl.kernel,
    out_type=jax.ShapeDtypeStruct((_NC, _NP), jnp.float32),
    mesh=_deg_mesh,
    scratch_types=[
        pltpu.VMEM((_CHUNK,), jnp.int32),
        pltpu.VMEM((_CHUNK,), jnp.float32),
        pltpu.VMEM_SHARED((_NP,), jnp.float32),
    ],
    compiler_params=pltpu.CompilerParams(use_tc_tiling_on_sc=False),
)
def _sc_degree(dst_hbm, out_hbm, dst_v, ones_v, acc_sh):
    cid = lax.axis_index("c")
    sid = lax.axis_index("s")

    one = jnp.full((_H,), 1.0, jnp.float32)
    zero = jnp.zeros((_H,), jnp.float32)
    for i in range(_DPT // _H):
        ones_v[pl.ds(i * _H, _H)] = zero
    pltpu.sync_copy(ones_v.at[pl.ds(0, _DPT)], acc_sh.at[pl.ds(sid * _DPT, _DPT)])
    for i in range(_CHUNK // _H):
        ones_v[pl.ds(i * _H, _H)] = one
    plsc.subcore_barrier()

    for c in range(_NCHUNK):
        base = cid * _EPC + sid * _EPT + c * _CHUNK
        pltpu.sync_copy(dst_hbm.at[pl.ds(base, _CHUNK)], dst_v)
        pltpu.sync_copy(ones_v, acc_sh.at[dst_v], add=True)

    plsc.subcore_barrier()
    pltpu.sync_copy(
        acc_sh.at[pl.ds(sid * _DPT, _DPT)],
        out_hbm.at[cid, pl.ds(sid * _DPT, _DPT)],
    )


_scat_mesh = plsc.VectorSubcoreMesh(core_axis_name="c", subcore_axis_name="s")


@functools.partial(
    pl.kernel,
    out_type=jax.ShapeDtypeStruct((_NC, _NP, _H), jnp.float32),
    mesh=_scat_mesh,
    scratch_types=[
        pltpu.VMEM((_CHUNK,), jnp.int32),
        pltpu.VMEM((_CHUNK,), jnp.int32),
        pltpu.VMEM((_CHUNK, _H), jnp.float32),
        pltpu.VMEM_SHARED((_NP, _H), jnp.float32),
        pltpu.SemaphoreType.DMA,
    ],
    compiler_params=pltpu.CompilerParams(use_tc_tiling_on_sc=False),
)
def _sc_scatter(u_hbm, src_hbm, dst_hbm, out_hbm, src_v, dst_v, rows_v, acc_sh, sem):
    cid = lax.axis_index("c")
    sid = lax.axis_index("s")

    _fill_zero_rows(rows_v, _RPT)
    pltpu.sync_copy(rows_v.at[pl.ds(0, _RPT)], acc_sh.at[pl.ds(sid * _RPT, _RPT)])
    plsc.subcore_barrier()

    for c in range(_NCHUNK):
        base = cid * _EPC + sid * _EPT + c * _CHUNK
        pltpu.sync_copy(src_hbm.at[pl.ds(base, _CHUNK)], src_v)
        pltpu.sync_copy(dst_hbm.at[pl.ds(base, _CHUNK)], dst_v)
        pltpu.async_copy(u_hbm.at[src_v], rows_v, sem).wait()
        pltpu.sync_copy(rows_v, acc_sh.at[dst_v], add=True)

    plsc.subcore_barrier()
    pltpu.sync_copy(
        acc_sh.at[pl.ds(sid * _RPT, _RPT)],
        out_hbm.at[cid, pl.ds(sid * _RPT, _RPT)],
    )


def _tc_mm1_body(x_ref, w_ref, o_ref):
    o_ref[...] = jnp.dot(x_ref[...], w_ref[...], preferred_element_type=jnp.float32)


def _tc_prescale_body(h_ref, degt_ref, u1_ref, dinv_ref):
    deg = degt_ref[:, 0:1] + degt_ref[:, 1:2] + 1.0
    dinv = lax.rsqrt(deg)
    dinv_ref[...] = dinv
    u1_ref[...] = h_ref[...] * dinv


def _tc_mid_body(s1_ref, u1_ref, dinv_ref, b1_ref, u2_ref):
    dinv = dinv_ref[...]
    s = s1_ref[0, 0:_N, :] + s1_ref[1, 0:_N, :] + u1_ref[...]
    z = s * dinv + jnp.reshape(b1_ref[...], (1, _H))
    u2_ref[...] = z * dinv


def _tc_final_body(s2_ref, u2_ref, dinv_ref, w2_ref, b2_ref, o_ref):
    m2 = (s2_ref[0, 0:_N, :] + s2_ref[1, 0:_N, :] + u2_ref[...]) * dinv_ref[...]
    o_ref[...] = (
        jnp.dot(m2, w2_ref[...], preferred_element_type=jnp.float32)
        + jnp.reshape(b2_ref[...], (1, _D))
    )


def kernel(x, edge_index, W1, b1, W2, b2):
    src = edge_index[0]
    dst = edge_index[1]

    degp = _sc_degree(dst)
    degt = jnp.transpose(degp)[:_N]

    h = pl.pallas_call(
        _tc_mm1_body,
        out_shape=jax.ShapeDtypeStruct((_N, _H), jnp.float32),
    )(x, W1)

    u1, dinv = pl.pallas_call(
        _tc_prescale_body,
        out_shape=(
            jax.ShapeDtypeStruct((_N, _H), jnp.float32),
            jax.ShapeDtypeStruct((_N, 1), jnp.float32),
        ),
    )(h, degt)

    s1 = _sc_scatter(u1, src, dst)

    u2 = pl.pallas_call(
        _tc_mid_body,
        out_shape=jax.ShapeDtypeStruct((_N, _H), jnp.float32),
    )(s1, u1, dinv, b1)

    s2 = _sc_scatter(u2, src, dst)

    out = pl.pallas_call(
        _tc_final_body,
        out_shape=jax.ShapeDtypeStruct((_N, _D), jnp.float32),
    )(s2, u2, dinv, W2, b2)
    return out


# speedup vs baseline: 68.3952x; 1.1462x over previous
import functools

import jax
import jax.numpy as jnp
from jax import lax
from jax.experimental import pallas as pl
from jax.experimental.pallas import tpu as pltpu
from jax.experimental.pallas import tpu_sc as plsc

_N = 10000
_E = 320000
_D = 128
_H = 16

_NC = 2
_NS = 16
_NW = _NC * _NS

_EPC = _E // _NC
_EPT = _E // _NW
_CHUNK = 2000
_NCHUNK = _EPT // _CHUNK
_SCHUNK = 2000
_NSCHUNK = _EPT // _SCHUNK

_NP = 10240
_RPT = _NP // _NS
_DPT = _NP // _NS


def _fill_zero_rows(ref, nrows):
    zero = jnp.zeros((_H,), jnp.float32)

    def body(i, _):
        ref[i, :] = zero
        return 0

    lax.fori_loop(0, nrows, body, 0)


_deg_mesh = plsc.VectorSubcoreMesh(core_axis_name="c", subcore_axis_name="s")


@functools.partial(
    pl.kernel,
    out_type=jax.ShapeDtypeStruct((_NC, _NP), jnp.float32),
    mesh=_deg_mesh,
    scratch_types=[
        pltpu.VMEM((_CHUNK,), jnp.int32),
        pltpu.VMEM((_CHUNK,), jnp.float32),
        pltpu.VMEM_SHARED((_NP,), jnp.float32),
    ],
    compiler_params=pltpu.CompilerParams(use_tc_tiling_on_sc=False),
)
def _sc_degree(dst_hbm, out_hbm, dst_v, ones_v, acc_sh):
    cid = lax.axis_index("c")
    sid = lax.axis_index("s")

    one = jnp.full((_H,), 1.0, jnp.float32)
    zero = jnp.zeros((_H,), jnp.float32)
    for i in range(_DPT // _H):
        ones_v[pl.ds(i * _H, _H)] = zero
    pltpu.sync_copy(ones_v.at[pl.ds(0, _DPT)], acc_sh.at[pl.ds(sid * _DPT, _DPT)])
    for i in range(_CHUNK // _H):
        ones_v[pl.ds(i * _H, _H)] = one
    plsc.subcore_barrier()

    for c in range(_NCHUNK):
        base = cid * _EPC + sid * _EPT + c * _CHUNK
        pltpu.sync_copy(dst_hbm.at[pl.ds(base, _CHUNK)], dst_v)
        pltpu.sync_copy(ones_v, acc_sh.at[dst_v], add=True)

    plsc.subcore_barrier()
    pltpu.sync_copy(
        acc_sh.at[pl.ds(sid * _DPT, _DPT)],
        out_hbm.at[cid, pl.ds(sid * _DPT, _DPT)],
    )


_scat_mesh = plsc.VectorSubcoreMesh(core_axis_name="c", subcore_axis_name="s")


@functools.partial(
    pl.kernel,
    out_type=jax.ShapeDtypeStruct((_NC, _NP, _H), jnp.float32),
    mesh=_scat_mesh,
    scratch_types=[
        pltpu.VMEM((_SCHUNK,), jnp.int32),
        pltpu.VMEM((_SCHUNK,), jnp.int32),
        pltpu.VMEM((_SCHUNK, _H), jnp.float32),
        pltpu.VMEM((_SCHUNK,), jnp.int32),
        pltpu.VMEM((_SCHUNK,), jnp.int32),
        pltpu.VMEM((_SCHUNK, _H), jnp.float32),
        pltpu.VMEM_SHARED((_NP, _H), jnp.float32),
        pltpu.SemaphoreType.DMA,
        pltpu.SemaphoreType.DMA,
        pltpu.SemaphoreType.DMA,
        pltpu.SemaphoreType.DMA,
        pltpu.SemaphoreType.DMA,
        pltpu.SemaphoreType.DMA,
    ],
    compiler_params=pltpu.CompilerParams(use_tc_tiling_on_sc=False),
)
def _sc_scatter(u_hbm, src_hbm, dst_hbm, out_hbm,
                src0, dst0, rows0, src1, dst1, rows1, acc_sh,
                sem_i0, sem_g0, sem_s0, sem_i1, sem_g1, sem_s1):
    cid = lax.axis_index("c")
    sid = lax.axis_index("s")

    bufs = ((src0, dst0, rows0, sem_i0, sem_g0, sem_s0),
            (src1, dst1, rows1, sem_i1, sem_g1, sem_s1))

    _fill_zero_rows(rows0, _RPT)
    pltpu.sync_copy(rows0.at[pl.ds(0, _RPT)], acc_sh.at[pl.ds(sid * _RPT, _RPT)])
    plsc.subcore_barrier()

    def start_idx(c):
        s_v, d_v = bufs[c % 2][0], bufs[c % 2][1]
        sem = bufs[c % 2][3]
        base = cid * _EPC + sid * _EPT + c * _SCHUNK
        c1 = pltpu.async_copy(src_hbm.at[pl.ds(base, _SCHUNK)], s_v, sem)
        c2 = pltpu.async_copy(dst_hbm.at[pl.ds(base, _SCHUNK)], d_v, sem)
        return (c1, c2)

    pend_idx = [None, None]
    pend_scat = [None, None]
    pend_idx[0] = start_idx(0)
    for c in range(_NSCHUNK):
        b = c % 2
        s_v, d_v, r_v, _, sem_g, sem_s = bufs[b]
        for d in pend_idx[b]:
            d.wait()
        gat = pltpu.async_copy(u_hbm.at[s_v], r_v, sem_g)
        if c + 1 < _NSCHUNK:
            if pend_scat[1 - b] is not None:
                pend_scat[1 - b].wait()
                pend_scat[1 - b] = None
            pend_idx[1 - b] = start_idx(c + 1)
        gat.wait()
        pend_scat[b] = pltpu.async_copy(r_v, acc_sh.at[d_v], sem_s, add=True)
    for b in range(2):
        if pend_scat[b] is not None:
            pend_scat[b].wait()

    plsc.subcore_barrier()
    pltpu.sync_copy(
        acc_sh.at[pl.ds(sid * _RPT, _RPT)],
        out_hbm.at[cid, pl.ds(sid * _RPT, _RPT)],
    )


def _tc_front_body(x_ref, w_ref, degt_ref, u1_ref, dinv_ref):
    h = jnp.dot(x_ref[...], w_ref[...], preferred_element_type=jnp.float32)
    deg = degt_ref[:, 0:1] + degt_ref[:, 1:2] + 1.0
    dinv = lax.rsqrt(deg)
    dinv_ref[...] = dinv
    u1_ref[...] = h * dinv


def _tc_mid_body(s1_ref, u1_ref, dinv_ref, b1_ref, u2_ref):
    dinv = dinv_ref[...]
    s = s1_ref[0, 0:_N, :] + s1_ref[1, 0:_N, :] + u1_ref[...]
    z = s * dinv + jnp.reshape(b1_ref[...], (1, _H))
    u2_ref[...] = z * dinv


def _tc_final_body(s2_ref, u2_ref, dinv_ref, w2_ref, b2_ref, o_ref):
    m2 = (s2_ref[0, 0:_N, :] + s2_ref[1, 0:_N, :] + u2_ref[...]) * dinv_ref[...]
    o_ref[...] = (
        jnp.dot(m2, w2_ref[...], preferred_element_type=jnp.float32)
        + jnp.reshape(b2_ref[...], (1, _D))
    )


def kernel(x, edge_index, W1, b1, W2, b2):
    src = edge_index[0]
    dst = edge_index[1]

    degp = _sc_degree(dst)
    degt = jnp.transpose(degp)[:_N]

    u1, dinv = pl.pallas_call(
        _tc_front_body,
        out_shape=(
            jax.ShapeDtypeStruct((_N, _H), jnp.float32),
            jax.ShapeDtypeStruct((_N, 1), jnp.float32),
        ),
    )(x, W1, degt)

    s1 = _sc_scatter(u1, src, dst)

    u2 = pl.pallas_call(
        _tc_mid_body,
        out_shape=jax.ShapeDtypeStruct((_N, _H), jnp.float32),
    )(s1, u1, dinv, b1)

    s2 = _sc_scatter(u2, src, dst)

    out = pl.pallas_call(
        _tc_final_body,
        out_shape=jax.ShapeDtypeStruct((_N, _D), jnp.float32),
    )(s2, u2, dinv, W2, b2)
    return out


# speedup vs baseline: 75.6361x; 1.1059x over previous
import functools

import jax
import jax.numpy as jnp
from jax import lax
from jax.experimental import pallas as pl
from jax.experimental.pallas import tpu as pltpu
from jax.experimental.pallas import tpu_sc as plsc

_N = 10000
_E = 320000
_D = 128
_H = 16

_NC = 2
_NS = 16
_NW = _NC * _NS

_EPC = _E // _NC
_EPT = _E // _NW
_CHUNK = 2000
_NCHUNK = _EPT // _CHUNK
_SCHUNK = 2000
_NSCHUNK = _EPT // _SCHUNK

_NP = 10240
_RPT = _NP // _NS
_DPT = _NP // _NS

_scat_scratch = [
    pltpu.VMEM((_SCHUNK,), jnp.int32),
    pltpu.VMEM((_SCHUNK,), jnp.int32),
    pltpu.VMEM((_SCHUNK, _H), jnp.float32),
    pltpu.VMEM((_SCHUNK,), jnp.int32),
    pltpu.VMEM((_SCHUNK,), jnp.int32),
    pltpu.VMEM((_SCHUNK, _H), jnp.float32),
    pltpu.SemaphoreType.DMA,
    pltpu.SemaphoreType.DMA,
    pltpu.SemaphoreType.DMA,
    pltpu.SemaphoreType.DMA,
    pltpu.SemaphoreType.DMA,
    pltpu.SemaphoreType.DMA,
]


def _edge_pipeline(gather_ref, src_hbm, dst_hbm, acc_sh, bufs, cid, sid):
    def start_idx(c):
        s_v, d_v = bufs[c % 2][0], bufs[c % 2][1]
        sem = bufs[c % 2][3]
        base = cid * _EPC + sid * _EPT + c * _SCHUNK
        c1 = pltpu.async_copy(src_hbm.at[pl.ds(base, _SCHUNK)], s_v, sem)
        c2 = pltpu.async_copy(dst_hbm.at[pl.ds(base, _SCHUNK)], d_v, sem)
        return (c1, c2)

    pend_idx = [None, None]
    pend_scat = [None, None]
    pend_idx[0] = start_idx(0)
    for c in range(_NSCHUNK):
        b = c % 2
        s_v, d_v, r_v, _, sem_g, sem_s = bufs[b]
        for d in pend_idx[b]:
            d.wait()
        gat = pltpu.async_copy(gather_ref.at[s_v], r_v, sem_g)
        if c + 1 < _NSCHUNK:
            if pend_scat[1 - b] is not None:
                pend_scat[1 - b].wait()
                pend_scat[1 - b] = None
            pend_idx[1 - b] = start_idx(c + 1)
        gat.wait()
        pend_scat[b] = pltpu.async_copy(r_v, acc_sh.at[d_v], sem_s, add=True)
    for b in range(2):
        if pend_scat[b] is not None:
            pend_scat[b].wait()


_deg_mesh = plsc.VectorSubcoreMesh(core_axis_name="c", subcore_axis_name="s")


@functools.partial(
    pl.kernel,
    out_type=jax.ShapeDtypeStruct((_NC, _NP), jnp.float32),
    mesh=_deg_mesh,
    scratch_types=[
        pltpu.VMEM((_CHUNK,), jnp.int32),
        pltpu.VMEM((_CHUNK,), jnp.float32),
        pltpu.VMEM_SHARED((_NP,), jnp.float32),
    ],
    compiler_params=pltpu.CompilerParams(use_tc_tiling_on_sc=False),
)
def _sc_degree(dst_hbm, out_hbm, dst_v, ones_v, acc_sh):
    cid = lax.axis_index("c")
    sid = lax.axis_index("s")

    one = jnp.full((_H,), 1.0, jnp.float32)
    zero = jnp.zeros((_H,), jnp.float32)
    for i in range(_DPT // _H):
        ones_v[pl.ds(i * _H, _H)] = zero
    pltpu.sync_copy(ones_v.at[pl.ds(0, _DPT)], acc_sh.at[pl.ds(sid * _DPT, _DPT)])
    for i in range(_CHUNK // _H):
        ones_v[pl.ds(i * _H, _H)] = one
    plsc.subcore_barrier()

    for c in range(_NCHUNK):
        base = cid * _EPC + sid * _EPT + c * _CHUNK
        pltpu.sync_copy(dst_hbm.at[pl.ds(base, _CHUNK)], dst_v)
        pltpu.sync_copy(ones_v, acc_sh.at[dst_v], add=True)

    plsc.subcore_barrier()
    pltpu.sync_copy(
        acc_sh.at[pl.ds(sid * _DPT, _DPT)],
        out_hbm.at[cid, pl.ds(sid * _DPT, _DPT)],
    )


_scat_mesh = plsc.VectorSubcoreMesh(core_axis_name="c", subcore_axis_name="s")


@functools.partial(
    pl.kernel,
    out_type=jax.ShapeDtypeStruct((_NC, _NP, _H), jnp.float32),
    mesh=_scat_mesh,
    scratch_types=_scat_scratch + [
        pltpu.VMEM_SHARED((_NP, _H), jnp.float32),
    ],
    compiler_params=pltpu.CompilerParams(use_tc_tiling_on_sc=False),
)
def _sc_pass1(u1p_hbm, src_hbm, dst_hbm, out_hbm,
              src0, dst0, rows0, src1, dst1, rows1,
              sem_i0, sem_g0, sem_s0, sem_i1, sem_g1, sem_s1, acc_sh):
    cid = lax.axis_index("c")
    sid = lax.axis_index("s")
    bufs = ((src0, dst0, rows0, sem_i0, sem_g0, sem_s0),
            (src1, dst1, rows1, sem_i1, sem_g1, sem_s1))

    pltpu.sync_copy(u1p_hbm.at[pl.ds(sid * _RPT, _RPT)], rows0.at[pl.ds(0, _RPT)])

    def scale_body(i, _):
        rows0[i, :] = rows0[i, :] * 0.5
        return 0

    lax.fori_loop(0, _RPT, scale_body, 0)
    pltpu.sync_copy(rows0.at[pl.ds(0, _RPT)], acc_sh.at[pl.ds(sid * _RPT, _RPT)])
    plsc.subcore_barrier()

    _edge_pipeline(u1p_hbm, src_hbm, dst_hbm, acc_sh, bufs, cid, sid)

    plsc.subcore_barrier()
    pltpu.sync_copy(
        acc_sh.at[pl.ds(sid * _RPT, _RPT)],
        out_hbm.at[cid, pl.ds(sid * _RPT, _RPT)],
    )


@functools.partial(
    pl.kernel,
    out_type=jax.ShapeDtypeStruct((_NC, _NP, _H), jnp.float32),
    mesh=_scat_mesh,
    scratch_types=_scat_scratch + [
        pltpu.VMEM_SHARED((_NP, _H), jnp.float32),
        pltpu.VMEM_SHARED((_NP, _H), jnp.float32),
    ],
    compiler_params=pltpu.CompilerParams(use_tc_tiling_on_sc=False),
)
def _sc_pass2(s1p_hbm, dsq_hbm, db_hbm, src_hbm, dst_hbm, out_hbm,
              src0, dst0, rows0, src1, dst1, rows1,
              sem_i0, sem_g0, sem_s0, sem_i1, sem_g1, sem_s1,
              u2_sh, acc_sh):
    cid = lax.axis_index("c")
    sid = lax.axis_index("s")
    bufs = ((src0, dst0, rows0, sem_i0, sem_g0, sem_s0),
            (src1, dst1, rows1, sem_i1, sem_g1, sem_s1))
    sl = pl.ds(sid * _RPT, _RPT)

    c1 = pltpu.async_copy(s1p_hbm.at[0, sl], rows0.at[pl.ds(0, _RPT)], sem_i0)
    c2 = pltpu.async_copy(s1p_hbm.at[1, sl], rows0.at[pl.ds(_RPT, _RPT)], sem_i0)
    c3 = pltpu.async_copy(dsq_hbm.at[sl], rows1.at[pl.ds(0, _RPT)], sem_i0)
    c4 = pltpu.async_copy(db_hbm.at[sl], rows1.at[pl.ds(_RPT, _RPT)], sem_i0)
    for c in (c1, c2, c3, c4):
        c.wait()

    def mid_body(i, _):
        u2 = (rows0[i, :] + rows0[_RPT + i, :]) * rows1[i, :] + rows1[_RPT + i, :]
        rows0[i, :] = u2
        rows1[i, :] = u2 * 0.5
        return 0

    lax.fori_loop(0, _RPT, mid_body, 0)
    pltpu.sync_copy(rows0.at[pl.ds(0, _RPT)], u2_sh.at[sl])
    pltpu.sync_copy(rows1.at[pl.ds(0, _RPT)], acc_sh.at[sl])
    plsc.subcore_barrier()

    _edge_pipeline(u2_sh, src_hbm, dst_hbm, acc_sh, bufs, cid, sid)

    plsc.subcore_barrier()
    pltpu.sync_copy(acc_sh.at[sl], out_hbm.at[cid, sl])


def _tc_front_body(x_ref, w_ref, degt_ref, b1_ref, u1p_ref, dsq_ref, db_ref,
                   dinv_ref):
    h = jnp.dot(x_ref[...], w_ref[...], preferred_element_type=jnp.float32)
    deg = degt_ref[:, 0:1] + degt_ref[:, 1:2] + 1.0
    dinv = lax.rsqrt(deg)
    dinv_ref[...] = dinv
    zpad = jnp.zeros((_NP - _N, _H), jnp.float32)
    u1p_ref[0:_N, :] = h * dinv
    u1p_ref[_N:_NP, :] = zpad
    dsq_ref[0:_N, :] = jnp.broadcast_to(dinv * dinv, (_N, _H))
    dsq_ref[_N:_NP, :] = zpad
    db_ref[0:_N, :] = jnp.broadcast_to(jnp.reshape(b1_ref[...], (1, _H)) * dinv,
                                       (_N, _H))
    db_ref[_N:_NP, :] = zpad


def _tc_final_body(s2_ref, dinv_ref, w2_ref, b2_ref, o_ref):
    m2 = (s2_ref[0, 0:_N, :] + s2_ref[1, 0:_N, :]) * dinv_ref[...]
    o_ref[...] = (
        jnp.dot(m2, w2_ref[...], preferred_element_type=jnp.float32)
        + jnp.reshape(b2_ref[...], (1, _D))
    )


def kernel(x, edge_index, W1, b1, W2, b2):
    src = edge_index[0]
    dst = edge_index[1]

    degp = _sc_degree(dst)
    degt = jnp.transpose(degp)[:_N]

    u1p, dsq, db, dinv = pl.pallas_call(
        _tc_front_body,
        out_shape=(
            jax.ShapeDtypeStruct((_NP, _H), jnp.float32),
            jax.ShapeDtypeStruct((_NP, _H), jnp.float32),
            jax.ShapeDtypeStruct((_NP, _H), jnp.float32),
            jax.ShapeDtypeStruct((_N, 1), jnp.float32),
        ),
    )(x, W1, degt, b1)

    s1p = _sc_pass1(u1p, src, dst)

    s2p = _sc_pass2(s1p, dsq, db, src, dst)

    out = pl.pallas_call(
        _tc_final_body,
        out_shape=jax.ShapeDtypeStruct((_N, _D), jnp.float32),
    )(s2p, dinv, W2, b2)
    return out
